# baseline (speedup 1.0000x reference)
import functools

import jax
import jax.numpy as jnp
from jax import lax
from jax.experimental import pallas as pl
from jax.experimental.pallas import tpu as pltpu
from jax.experimental.pallas import tpu_sc as plsc

MAX_LEN = 50
BATCH = 16384
EMBED = 32
VOCAB = 1000000
TOTAL = MAX_LEN * BATCH
NUM_WORKERS = 32

VMAIN = 999936
VTAIL = VOCAB - VMAIN
CHV = 512
NCHV = VMAIN // CHV
K1_GROUPS = 31

PER_WORKER = TOTAL // NUM_WORKERS
CHUNK = 512
CPT = BATCH // CHUNK
NCHUNK = PER_WORKER // CHUNK
NBUF = 5
NGROUP = NCHUNK // NBUF

_mesh = plsc.VectorSubcoreMesh(core_axis_name="c", subcore_axis_name="s")


PITCH = 33


@functools.partial(
    pl.kernel,
    out_type=jax.ShapeDtypeStruct((VOCAB * PITCH,), jnp.float32),
    mesh=_mesh,
    compiler_params=pltpu.CompilerParams(
        use_tc_tiling_on_sc=True, needs_layout_passes=False),
    scratch_types=[
        pltpu.VMEM((EMBED, CHV), jnp.float32),
        pltpu.VMEM((EMBED, CHV), jnp.float32),
        pltpu.VMEM((CHV * PITCH,), jnp.float32),
        pltpu.VMEM((CHV * PITCH,), jnp.float32),
        pltpu.VMEM((VTAIL, EMBED), jnp.float32),
        pltpu.VMEM((VTAIL * PITCH,), jnp.float32),
        [pltpu.SemaphoreType.DMA] * 2,
        [pltpu.SemaphoreType.DMA] * 2,
    ],
)
def _table_to_rowmajor(tab_t, tail, out_flat, colbuf0, colbuf1, rowbuf0,
                       rowbuf1, tailbuf, tailrow, isems, osems):
    colbufs = (colbuf0, colbuf1)
    rowbufs = (rowbuf0, rowbuf1)
    wid = lax.axis_index("s") * 2 + lax.axis_index("c")
    iota16 = lax.iota(jnp.int32, 16)
    iota_p = iota16 * PITCH

    def chunk_id(go, p):
        return wid + (go * 2 + p) * NUM_WORKERS

    def fire_in(chunk, p):
        pltpu.async_copy(
            tab_t.at[:, pl.ds(chunk * CHV, CHV)], colbufs[p], isems[p])

    def wait_in(p):
        pltpu.make_async_copy(
            tab_t.at[:, pl.ds(0, CHV)], colbufs[p], isems[p]).wait()

    def fire_out(chunk, p):
        pltpu.async_copy(
            rowbufs[p],
            out_flat.at[pl.ds(chunk * CHV * PITCH, CHV * PITCH)], osems[p])

    def wait_out(p):
        pltpu.make_async_copy(
            rowbufs[p], out_flat.at[pl.ds(0, CHV * PITCH)], osems[p]).wait()

    def compute(p):
        def xpose(w16, carry):
            off16 = w16 * 16

            for c in range(EMBED):
                v = colbufs[p][c, pl.ds(off16, 16)]
                plsc.store_scatter(
                    rowbufs[p], [iota_p + (off16 * PITCH + c)], v)
            return carry

        lax.fori_loop(0, CHV // 16, xpose, 0)

    for p in range(2):
        fire_in(chunk_id(0, p), p)

    def body(go, carry):
        for p in range(2):
            wait_in(p)
            compute(p)
            fire_out(chunk_id(go, p), p)
        for p in range(2):
            nxt = chunk_id(go + 1, p)
            wait_out(p)

            @pl.when(nxt < NCHV)
            def _():
                fire_in(nxt, p)

        return carry

    lax.fori_loop(0, K1_GROUPS - 1, body, 0)

    wait_in(0)
    compute(0)
    fire_out(chunk_id(K1_GROUPS - 1, 0), 0)

    @pl.when(chunk_id(K1_GROUPS - 1, 1) < NCHV)
    def _lastp1():
        wait_in(1)
        compute(1)
        fire_out(chunk_id(K1_GROUPS - 1, 1), 1)

    wait_out(0)

    @pl.when(chunk_id(K1_GROUPS - 1, 1) < NCHV)
    def _drain1():
        wait_out(1)

    @pl.when(wid == 0)
    def _tail():
        pltpu.async_copy(tail, tailbuf, isems[0]).wait()
        for r in range(VTAIL):
            for h in range(EMBED // 16):
                v = tailbuf[r, pl.ds(h * 16, 16)]
                plsc.store_scatter(
                    tailrow, [iota16 + (r * PITCH + h * 16)], v)
        pltpu.async_copy(
            tailrow, out_flat.at[pl.ds(VMAIN * PITCH, VTAIL * PITCH)],
            isems[0]).wait()


@functools.partial(
    pl.kernel,
    out_type=jax.ShapeDtypeStruct((MAX_LEN, BATCH, EMBED), jnp.float32),
    mesh=_mesh,
    compiler_params=pltpu.CompilerParams(use_tc_tiling_on_sc=False),
    scratch_types=[
        pltpu.VMEM((PER_WORKER,), jnp.int32),
        pltpu.VMEM((NBUF, CHUNK, PITCH), jnp.float32),
        [pltpu.SemaphoreType.DMA] * NBUF,
        [pltpu.SemaphoreType.DMA] * NBUF,
    ],
)
def _embedding_gather(idx_hbm, table_hbm, out_hbm, idx_v, rows_v, gsems, ssems):
    wid = lax.axis_index("s") * 2 + lax.axis_index("c")
    base = wid * PER_WORKER
    cbase = wid * NCHUNK
    pltpu.sync_copy(idx_hbm.at[pl.ds(base, PER_WORKER)], idx_v)

    def fire_gather(g, b):
        pltpu.async_copy(
            table_hbm.at[idx_v.at[pl.ds(g * CHUNK, CHUNK)]],
            rows_v.at[b], gsems[b])

    def wait_gather(b):
        pltpu.make_async_copy(
            table_hbm.at[idx_v.at[pl.ds(0, CHUNK)]],
            rows_v.at[b], gsems[b]).wait()

    def fire_store(g, b):
        c = cbase + g
        t = c // CPT
        b0 = (c % CPT) * CHUNK
        pltpu.async_copy(
            rows_v.at[b, :, pl.ds(0, EMBED)],
            out_hbm.at[t, pl.ds(b0, CHUNK), :], ssems[b])

    def wait_store(b):
        pltpu.make_async_copy(
            rows_v.at[b, :, pl.ds(0, EMBED)],
            out_hbm.at[0, pl.ds(0, CHUNK), :], ssems[b]).wait()

    for b in range(NBUF):
        fire_gather(b, b)

    def body(go, carry):
        for b in range(NBUF):
            wait_gather(b)
            fire_store(go * NBUF + b, b)
        for b in range(NBUF):
            wait_store(b)
            fire_gather((go + 1) * NBUF + b, b)
        return carry

    lax.fori_loop(0, NGROUP - 1, body, 0)

    last = (NGROUP - 1) * NBUF
    for b in range(NBUF):
        wait_gather(b)
        fire_store(last + b, b)
    for b in range(NBUF):
        wait_store(b)


def kernel(inputs, src_emb, tgt_emb):
    del tgt_emb
    flat_idx = inputs.reshape(TOTAL)
    tab_flat = _table_to_rowmajor(src_emb.T, src_emb[VMAIN:])
    tab_rm = tab_flat.reshape(VOCAB, PITCH)
    return _embedding_gather(flat_idx, tab_rm)


# speedup vs baseline: 2.3467x; 2.3467x over previous
import functools

import jax
import jax.numpy as jnp
from jax import lax
from jax.experimental import pallas as pl
from jax.experimental.pallas import tpu as pltpu
from jax.experimental.pallas import tpu_sc as plsc

MAX_LEN = 50
BATCH = 16384
EMBED = 32
TOTAL = MAX_LEN * BATCH
NUM_WORKERS = 32
PER_WORKER = TOTAL // NUM_WORKERS
CHUNK = 640
NCHUNK = PER_WORKER // CHUNK
NBUF = 4
NGROUP = NCHUNK // NBUF

_mesh = plsc.VectorSubcoreMesh(core_axis_name="c", subcore_axis_name="s")


@functools.partial(
    pl.kernel,
    out_type=jax.ShapeDtypeStruct((TOTAL, EMBED), jnp.float32),
    mesh=_mesh,
    compiler_params=pltpu.CompilerParams(use_tc_tiling_on_sc=False),
    scratch_types=[
        pltpu.VMEM((PER_WORKER,), jnp.int32),
        pltpu.VMEM((NBUF, CHUNK, EMBED), jnp.float32),
        [pltpu.SemaphoreType.DMA] * NBUF,
        [pltpu.SemaphoreType.DMA] * NBUF,
    ],
)
def _embedding_gather(idx_hbm, table_hbm, out_hbm, idx_v, rows_v, gsems, ssems):
    wid = lax.axis_index("s") * 2 + lax.axis_index("c")
    base = wid * PER_WORKER
    pltpu.sync_copy(idx_hbm.at[pl.ds(base, PER_WORKER)], idx_v)

    def fire_gather(g, b):
        pltpu.async_copy(
            table_hbm.at[idx_v.at[pl.ds(g * CHUNK, CHUNK)]],
            rows_v.at[b], gsems[b])

    def wait_gather(b):
        pltpu.make_async_copy(
            table_hbm.at[idx_v.at[pl.ds(0, CHUNK)]],
            rows_v.at[b], gsems[b]).wait()

    def fire_store(g, b):
        pltpu.async_copy(
            rows_v.at[b], out_hbm.at[pl.ds(base + g * CHUNK, CHUNK)], ssems[b])

    def wait_store(b):
        pltpu.make_async_copy(
            rows_v.at[b], out_hbm.at[pl.ds(base, CHUNK)], ssems[b]).wait()

    for b in range(NBUF):
        fire_gather(b, b)

    def body(go, carry):
        for b in range(NBUF):
            wait_gather(b)
            fire_store(go * NBUF + b, b)
        for b in range(NBUF):
            wait_store(b)
            fire_gather((go + 1) * NBUF + b, b)
        return carry

    lax.fori_loop(0, NGROUP - 1, body, 0)

    last = (NGROUP - 1) * NBUF
    for b in range(NBUF):
        wait_gather(b)
        fire_store(last + b, b)
    for b in range(NBUF):
        wait_store(b)


def kernel(inputs, src_emb, tgt_emb):
    del tgt_emb
    flat_idx = inputs.reshape(TOTAL)
    out = _embedding_gather(flat_idx, src_emb)
    return out.reshape(MAX_LEN, BATCH, EMBED)


# speedup vs baseline: 2.5199x; 1.0738x over previous
import functools

import jax
import jax.numpy as jnp
from jax import lax
from jax.experimental import pallas as pl
from jax.experimental.pallas import tpu as pltpu
from jax.experimental.pallas import tpu_sc as plsc

MAX_LEN = 50
BATCH = 16384
EMBED = 32
VOCAB = 1000000
TOTAL = MAX_LEN * BATCH
NUM_WORKERS = 32

VMAIN = 999936
VTAIL = VOCAB - VMAIN
CHV = 512
NCHV = VMAIN // CHV
K1_GROUPS = 31
PITCH = CHV + 1

PER_WORKER = TOTAL // NUM_WORKERS
CHUNK = 640
NCHUNK = PER_WORKER // CHUNK
NBUF = 4
NGROUP = NCHUNK // NBUF

_mesh = plsc.VectorSubcoreMesh(core_axis_name="c", subcore_axis_name="s")


@functools.partial(
    pl.kernel,
    out_type=jax.ShapeDtypeStruct((VOCAB * EMBED,), jnp.float32),
    mesh=_mesh,
    compiler_params=pltpu.CompilerParams(
        use_tc_tiling_on_sc=True, needs_layout_passes=False),
    scratch_types=[
        pltpu.VMEM((EMBED, CHV), jnp.float32),
        pltpu.VMEM((EMBED, CHV), jnp.float32),
        pltpu.VMEM((CHV * EMBED,), jnp.float32),
        pltpu.VMEM((CHV * EMBED,), jnp.float32),
        pltpu.VMEM((EMBED * PITCH,), jnp.float32),
        pltpu.VMEM((VTAIL, EMBED), jnp.float32),
        pltpu.VMEM((VTAIL * EMBED,), jnp.float32),
        [pltpu.SemaphoreType.DMA] * 2,
        [pltpu.SemaphoreType.DMA] * 2,
    ],
)
def _table_to_rowmajor(tab_t, tail, out_flat, colbuf0, colbuf1, rowbuf0,
                       rowbuf1, oddbuf, tailbuf, tailrow, isems, osems):
    colbufs = (colbuf0, colbuf1)
    rowbufs = (rowbuf0, rowbuf1)
    wid = lax.axis_index("s") * 2 + lax.axis_index("c")
    iota16 = lax.iota(jnp.int32, 16)
    iota_pitch = iota16 * PITCH

    def chunk_id(go, p):
        return wid + (go * 2 + p) * NUM_WORKERS

    def fire_in(chunk, p):
        pltpu.async_copy(
            tab_t.at[:, pl.ds(chunk * CHV, CHV)], colbufs[p], isems[p])

    def wait_in(p):
        pltpu.make_async_copy(
            tab_t.at[:, pl.ds(0, CHV)], colbufs[p], isems[p]).wait()

    def fire_out(chunk, p):
        pltpu.async_copy(
            rowbufs[p],
            out_flat.at[pl.ds(chunk * CHV * EMBED, CHV * EMBED)], osems[p])

    def wait_out(p):
        pltpu.make_async_copy(
            rowbufs[p], out_flat.at[pl.ds(0, CHV * EMBED)], osems[p]).wait()

    def compute(p):
        def pass1(w16, carry):
            off16 = w16 * 16
            for c in range(EMBED):
                oddbuf[pl.ds(c * PITCH + off16, 16)] = (
                    colbufs[p][c, pl.ds(off16, 16)])
            return carry

        lax.fori_loop(0, CHV // 16, pass1, 0)

        def pass2(rr, carry):
            r0 = rr * 8
            for dr in range(8):
                r = r0 + dr
                idx = iota_pitch + r
                g0 = plsc.load_gather(oddbuf, [idx])
                g1 = plsc.load_gather(oddbuf, [idx + 16 * PITCH])
                rowbufs[p][pl.ds(r * EMBED, 16)] = g0
                rowbufs[p][pl.ds(r * EMBED + 16, 16)] = g1
            return carry

        lax.fori_loop(0, CHV // 8, pass2, 0)

    for p in range(2):
        fire_in(chunk_id(0, p), p)

    def body(go, carry):
        for p in range(2):
            wait_in(p)
            compute(p)
            fire_out(chunk_id(go, p), p)
        for p in range(2):
            nxt = chunk_id(go + 1, p)
            wait_out(p)

            @pl.when(nxt < NCHV)
            def _():
                fire_in(nxt, p)

        return carry

    lax.fori_loop(0, K1_GROUPS - 1, body, 0)

    wait_in(0)
    compute(0)
    fire_out(chunk_id(K1_GROUPS - 1, 0), 0)

    @pl.when(chunk_id(K1_GROUPS - 1, 1) < NCHV)
    def _lastp1():
        wait_in(1)
        compute(1)
        fire_out(chunk_id(K1_GROUPS - 1, 1), 1)

    wait_out(0)

    @pl.when(chunk_id(K1_GROUPS - 1, 1) < NCHV)
    def _drain1():
        wait_out(1)

    @pl.when(wid == 0)
    def _tail():
        pltpu.async_copy(tail, tailbuf, isems[0]).wait()
        for r in range(VTAIL):
            for h in range(EMBED // 16):
                v = tailbuf[r, pl.ds(h * 16, 16)]
                plsc.store_scatter(
                    tailrow, [iota16 + (r * EMBED + h * 16)], v)
        pltpu.async_copy(
            tailrow, out_flat.at[pl.ds(VMAIN * EMBED, VTAIL * EMBED)],
            isems[0]).wait()


@functools.partial(
    pl.kernel,
    out_type=jax.ShapeDtypeStruct((TOTAL, EMBED), jnp.float32),
    mesh=_mesh,
    compiler_params=pltpu.CompilerParams(use_tc_tiling_on_sc=False),
    scratch_types=[
        pltpu.VMEM((PER_WORKER,), jnp.int32),
        pltpu.VMEM((NBUF, CHUNK, EMBED), jnp.float32),
        [pltpu.SemaphoreType.DMA] * NBUF,
        [pltpu.SemaphoreType.DMA] * NBUF,
    ],
)
def _embedding_gather(idx_hbm, table_hbm, out_hbm, idx_v, rows_v, gsems, ssems):
    wid = lax.axis_index("s") * 2 + lax.axis_index("c")
    base = wid * PER_WORKER
    pltpu.sync_copy(idx_hbm.at[pl.ds(base, PER_WORKER)], idx_v)

    def fire_gather(g, b):
        pltpu.async_copy(
            table_hbm.at[idx_v.at[pl.ds(g * CHUNK, CHUNK)]],
            rows_v.at[b], gsems[b])

    def wait_gather(b):
        pltpu.make_async_copy(
            table_hbm.at[idx_v.at[pl.ds(0, CHUNK)]],
            rows_v.at[b], gsems[b]).wait()

    def fire_store(g, b):
        pltpu.async_copy(
            rows_v.at[b], out_hbm.at[pl.ds(base + g * CHUNK, CHUNK)], ssems[b])

    def wait_store(b):
        pltpu.make_async_copy(
            rows_v.at[b], out_hbm.at[pl.ds(base, CHUNK)], ssems[b]).wait()

    for b in range(NBUF):
        fire_gather(b, b)

    def body(go, carry):
        for b in range(NBUF):
            wait_gather(b)
            fire_store(go * NBUF + b, b)
        for b in range(NBUF):
            wait_store(b)
            fire_gather((go + 1) * NBUF + b, b)
        return carry

    lax.fori_loop(0, NGROUP - 1, body, 0)

    last = (NGROUP - 1) * NBUF
    for b in range(NBUF):
        wait_gather(b)
        fire_store(last + b, b)
    for b in range(NBUF):
        wait_store(b)


def kernel(inputs, src_emb, tgt_emb):
    del tgt_emb
    flat_idx = inputs.reshape(TOTAL)
    tab_flat = _table_to_rowmajor(src_emb.T, src_emb[VMAIN:])
    tab_rm = tab_flat.reshape(VOCAB, EMBED)
    out = _embedding_gather(flat_idx, tab_rm)
    return out.reshape(MAX_LEN, BATCH, EMBED)
